# baseline (speedup 1.0000x reference)
import functools

import jax
import jax.numpy as jnp
from jax import lax
from jax.experimental import pallas as pl
from jax.experimental.pallas import tpu as pltpu
from jax.experimental.pallas import tpu_sc as plsc

NC = 2
NS = 16
NW = NC * NS
CHUNK = 128


def _leaky(y):
    return jnp.where(y >= 0, y, 0.01 * y)


NB = 2
NI = 4


@functools.lru_cache(maxsize=None)
def _make_segsum(n_nodes, d, e_pad, gpad):
    kpt = e_pad // (NW * CHUNK)
    assert kpt > 2 * NI and (kpt - 4) % NI == 0
    n_acc = ((n_nodes + 1 + 8 * NS - 1) // (8 * NS)) * (8 * NS)
    zrows = n_acc // NS
    mesh = plsc.VectorSubcoreMesh(core_axis_name="c", subcore_axis_name="s")

    @functools.partial(
        pl.kernel,
        out_type=[
            jax.ShapeDtypeStruct((NC, n_acc, d), jnp.float32),
            jax.ShapeDtypeStruct((gpad, d), jnp.float32),
        ],
        mesh=mesh,
        scratch_types=[
            pltpu.VMEM((NI, 2, CHUNK), jnp.int32),
            pltpu.VMEM((NB, CHUNK, d), jnp.float32),
            pltpu.VMEM_SHARED((n_acc, d), jnp.float32),
            [pltpu.SemaphoreType.DMA] * NI,
            [pltpu.SemaphoreType.DMA] * NB,
            [pltpu.SemaphoreType.DMA] * NB,
            pltpu.SemaphoreType.DMA,
            pltpu.VMEM((gpad,), jnp.int32),
        ],
    )
    def segsum(idx_hbm, feat_hbm, zeros_hbm, bbox_hbm,
               out_hbm, gfeat_hbm,
               idxr, rows, acc, isems, gsems, ssems, sem, bidx):
        c = lax.axis_index("c")
        s = lax.axis_index("s")
        wid = s * NC + c

        pltpu.sync_copy(zeros_hbm, acc.at[pl.ds(s * zrows, zrows)])
        plsc.subcore_barrier()

        def start_idx(k, islot):
            pltpu.async_copy(idx_hbm.at[wid * kpt + k], idxr.at[islot],
                             isems[islot])

        def start_gather(islot, rslot):
            pltpu.async_copy(feat_hbm.at[idxr.at[islot, 0]], rows.at[rslot],
                             gsems[rslot])

        def start_scatter(islot, rslot):
            pltpu.async_copy(rows.at[rslot], acc.at[idxr.at[islot, 1]],
                             ssems[rslot], add=True)

        def wait_rows(semref):
            pltpu.make_async_copy(feat_hbm.at[pl.ds(0, CHUNK)],
                                  rows.at[0], semref).wait()

        def wait_idx(semref):
            pltpu.make_async_copy(idx_hbm.at[0], idxr.at[0], semref).wait()

        def step(t, tm, do_idx=True, do_gather=True, do_ssw=True,
                 do_scatter=True):
            if do_idx:
                start_idx(t + 2, (tm + 2) % NI)
            if do_gather:
                if do_ssw:
                    wait_rows(ssems[(tm + 1) % NB])
                wait_idx(isems[(tm + 1) % NI])
                start_gather((tm + 1) % NI, (tm + 1) % NB)
            if do_scatter:
                wait_rows(gsems[tm % NB])
                start_scatter(tm % NI, tm % NB)

        start_idx(0, 0)
        start_idx(1, 1)
        wait_idx(isems[0])
        start_gather(0, 0)
        step(0, 0, do_ssw=False)

        def body(j, carry):
            for b in range(NI):
                step(1 + j * NI + b, 1 + b)
            return carry

        lax.fori_loop(0, (kpt - 4) // NI, body, 0)
        step(kpt - 3, (kpt - 3) % NI)
        step(kpt - 2, (kpt - 2) % NI, do_idx=False)
        step(kpt - 1, (kpt - 1) % NI, do_idx=False, do_gather=False)
        wait_rows(ssems[(kpt - 2) % NB])
        wait_rows(ssems[(kpt - 1) % NB])

        plsc.subcore_barrier()
        pltpu.sync_copy(acc.at[pl.ds(s * zrows, zrows)],
                        out_hbm.at[c, pl.ds(s * zrows, zrows)])

        @pl.when(wid == 0)
        def _():
            pltpu.sync_copy(bbox_hbm, bidx)
            pltpu.async_copy(feat_hbm.at[bidx],
                             rows.at[0, pl.ds(0, gpad)], sem).wait()
            pltpu.sync_copy(rows.at[0, pl.ds(0, gpad)], gfeat_hbm)

    return segsum


@functools.lru_cache(maxsize=None)
def _make_gatherq(n_nodes, d, gpad):
    mesh = plsc.VectorSubcoreMesh(core_axis_name="c", subcore_axis_name="s")

    @functools.partial(
        pl.kernel,
        out_type=jax.ShapeDtypeStruct((NC, gpad, d), jnp.float32),
        mesh=mesh,
        scratch_types=[
            pltpu.VMEM((gpad,), jnp.int32),
            pltpu.VMEM((gpad, d), jnp.float32),
            pltpu.SemaphoreType.DMA,
        ],
    )
    def gatherq(q_hbm, bbox_hbm, out_hbm, bidx, brows, sem):
        c = lax.axis_index("c")
        s = lax.axis_index("s")
        wid = s * NC + c

        @pl.when(wid < NC)
        def _():
            pltpu.sync_copy(bbox_hbm, bidx)
            pltpu.async_copy(q_hbm.at[wid].at[bidx], brows, sem).wait()
            pltpu.sync_copy(brows, out_hbm.at[wid])

    return gatherq


def _tc1_body(p_ref, x_ref, wrel_ref, b_ref, wroot_ref, o_ref):
    agg = p_ref[0] + p_ref[1]
    y = lax.dot_general(agg, wrel_ref[...], (((1,), (1,)), ((), ())),
                        preferred_element_type=jnp.float32)
    y = y + b_ref[...] + lax.dot_general(
        x_ref[...], wroot_ref[...], (((1,), (1,)), ((), ())),
        preferred_element_type=jnp.float32)
    o_ref[...] = _leaky(y)


def _tc2_body(gq_ref, gh_ref, wrel_ref, b_ref, wroot_ref, o_ref):
    agg = gq_ref[0] + gq_ref[1]
    y = lax.dot_general(agg, wrel_ref[...], (((1,), (1,)), ((), ())),
                        preferred_element_type=jnp.float32)
    y = y + b_ref[...] + lax.dot_general(
        gh_ref[...], wroot_ref[...], (((1,), (1,)), ((), ())),
        preferred_element_type=jnp.float32)
    o_ref[...] = _leaky(y)


def kernel(x, edge_index, bbox, W1_rel, b1, W1_root, W2_rel, b2, W2_root):
    n, d = x.shape
    e = edge_index.shape[1]
    r = bbox.shape[0]

    quantum = NW * CHUNK * NB
    e_pad = ((e + quantum - 1) // quantum) * quantum
    gpad = ((r + 7) // 8) * 8

    src = edge_index[0]
    dst = edge_index[1]
    src_p = jnp.concatenate(
        [src, jnp.zeros((e_pad - e,), jnp.int32)]).reshape(-1, CHUNK)
    dst_p = jnp.concatenate(
        [dst, jnp.full((e_pad - e,), n, jnp.int32)]).reshape(-1, CHUNK)
    idx2 = jnp.stack([src_p, dst_p], axis=1)
    bbox_p = jnp.concatenate([bbox, jnp.zeros((gpad - r,), jnp.int32)])

    n_acc = ((n + 1 + 8 * NS - 1) // (8 * NS)) * (8 * NS)
    zeros_hbm = jnp.zeros((n_acc // NS, d), jnp.float32)
    b1_2d = b1.reshape(1, d)
    b2_2d = b2.reshape(1, d)

    segsum = _make_segsum(n, d, e_pad, gpad)
    gatherq = _make_gatherq(n, d, gpad)

    p1, _ = segsum(idx2, x, zeros_hbm, bbox_p)

    rb = 2000
    h = pl.pallas_call(
        _tc1_body,
        grid=(n // rb,),
        in_specs=[
            pl.BlockSpec((NC, rb, d), lambda i: (0, i, 0)),
            pl.BlockSpec((rb, d), lambda i: (i, 0)),
            pl.BlockSpec((d, d), lambda i: (0, 0)),
            pl.BlockSpec((1, d), lambda i: (0, 0)),
            pl.BlockSpec((d, d), lambda i: (0, 0)),
        ],
        out_specs=pl.BlockSpec((rb, d), lambda i: (i, 0)),
        out_shape=jax.ShapeDtypeStruct((n, d), jnp.float32),
    )(p1, x, W1_rel, b1_2d, W1_root)

    p2, gh = segsum(idx2, h, zeros_hbm, bbox_p)

    gq = gatherq(p2, bbox_p)

    out = pl.pallas_call(
        _tc2_body,
        out_shape=jax.ShapeDtypeStruct((gpad, d), jnp.float32),
    )(gq, gh, W2_rel, b2_2d, W2_root)

    return out[:r]
